# Initial kernel scaffold; baseline (speedup 1.0000x reference)
import jax
import jax.numpy as jnp
from jax.experimental import pallas as pl


def kernel(x):
    raise NotImplementedError("write your pallas kernel here")


# speedup vs baseline: 1.5957x; 1.5957x over previous
import functools

import jax
import jax.numpy as jnp
from jax.experimental import pallas as pl
from jax.experimental.pallas import tpu as pltpu

MD = 7
PATCH = 32
NUM_MA = 24


def _diff_kernel(x_ref, out_ref, acc_ref):
    c = pl.program_id(1)
    nc = pl.num_programs(1)

    part = jnp.abs(x_ref[0, 0, 1] - x_ref[0, 0, 0])
    for t in range(1, x_ref.shape[2] - 1):
        part = part + jnp.abs(x_ref[0, 0, t + 1] - x_ref[0, 0, t])

    @pl.when(c == 0)
    def _():
        acc_ref[...] = part

    @pl.when(c != 0)
    def _():
        acc_ref[...] = acc_ref[...] + part

    @pl.when(c == nc - 1)
    def _():
        acc = acc_ref[...]
        cols = jnp.concatenate(
            [acc[:, j * PATCH:(j + 1) * PATCH].sum(axis=1, keepdims=True)
             for j in range(MD)], axis=1)
        ps = jnp.concatenate(
            [cols[i * PATCH:(i + 1) * PATCH, :].sum(axis=0, keepdims=True)
             for i in range(MD)], axis=0)

        idx = jax.lax.broadcasted_iota(jnp.int32, (MD, MD), 0) * MD + \
              jax.lax.broadcasted_iota(jnp.int32, (MD, MD), 1)
        a = ps[:, :, None, None]
        b = ps[None, None, :, :]
        ia = idx[:, :, None, None]
        ib = idx[None, None, :, :]
        beats = (b > a) | ((b == a) & (ib < ia))
        rank = beats.astype(jnp.int32).sum(axis=(2, 3))
        out_ref[0] = (rank < NUM_MA).astype(jnp.float32)


@jax.jit
def kernel(x):
    B, C, T, H, W = x.shape
    return pl.pallas_call(
        _diff_kernel,
        grid=(B, C),
        in_specs=[pl.BlockSpec((1, 1, T, H, W), lambda b, c: (b, c, 0, 0, 0))],
        out_specs=pl.BlockSpec((1, MD, MD), lambda b, c: (b, 0, 0)),
        out_shape=jax.ShapeDtypeStruct((B, MD, MD), jnp.float32),
        scratch_shapes=[pltpu.VMEM((H, W), jnp.float32)],
    )(x)
